# baseline (speedup 1.0000x reference)
import functools

import jax
import jax.numpy as jnp
from jax import lax
from jax.experimental import pallas as pl
from jax.experimental.pallas import tpu as pltpu
from jax.experimental.pallas import tpu_sc as plsc

N = 10000
NP = 10240
E = 320000
H = 128
T = 3
L = 3

NUM_CORES = 2
NUM_SUBCORES = 16
NUM_WORKERS = NUM_CORES * NUM_SUBCORES
GC = 48
GSLOTS = 6
SSLICES = 3
SG = 2 * GC
GROUPS = 112
DHALF = 64
CHUNKS = 2 * GROUPS
EPW_P = GROUPS * SG
EP = NUM_WORKERS * EPW_P
ROWS_PER_TILE = NP // NUM_SUBCORES
FLUSH_CHUNK = 128


def _make_sc_agg(num_towers: int):
    mesh = plsc.VectorSubcoreMesh(core_axis_name="c", subcore_axis_name="s")

    def body(h_hbm, src_hbm, dst_hbm, zeros_hbm, out_hbm,
             i0, i1, i2, i3, i4, i5, dst_blk, ring, acc, *sems):
        c = lax.axis_index("c")
        s = lax.axis_index("s")
        wid = c * NUM_SUBCORES + s
        ibufs = (i0, i1, i2, i3, i4, i5)
        gsems = sems[:GSLOTS]
        isems = sems[GSLOTS:2 * GSLOTS]
        ssems = sems[2 * GSLOTS:]

        for t in range(num_towers):
            pltpu.sync_copy(zeros_hbm, ring.at[pl.ds(0, FLUSH_CHUNK)])
            r0 = s * ROWS_PER_TILE
            for k in range(ROWS_PER_TILE // FLUSH_CHUNK):
                pltpu.sync_copy(
                    ring.at[pl.ds(0, FLUSH_CHUNK)],
                    acc.at[pl.ds(r0 + k * FLUSH_CHUNK, FLUSH_CHUNK)])
            plsc.subcore_barrier()
            pltpu.sync_copy(dst_hbm.at[wid, pl.ds(0, DHALF)], dst_blk)

            table = h_hbm.at[t]

            def fire_i(ch, q):
                pltpu.async_copy(src_hbm.at[wid, ch], ibufs[q], isems[q])

            def wait_i(ch, q):
                pltpu.make_async_copy(src_hbm.at[wid, ch], ibufs[q],
                                      isems[q]).wait()

            def fire_g(q):
                pltpu.async_copy(table.at[ibufs[q]],
                                 ring.at[pl.ds(q * GC, GC)], gsems[q])

            def wait_g(q):
                pltpu.make_async_copy(table.at[ibufs[q]],
                                      ring.at[pl.ds(q * GC, GC)],
                                      gsems[q]).wait()

            def fire_s(row, sl):
                pltpu.async_copy(ring.at[pl.ds(2 * sl * GC, SG)],
                                 acc.at[dst_blk.at[row]], ssems[sl],
                                 add=True)

            def wait_s(row, sl):
                pltpu.make_async_copy(ring.at[pl.ds(2 * sl * GC, SG)],
                                      acc.at[dst_blk.at[row]],
                                      ssems[sl]).wait()

            def group_body(g, sl, doff, wait_prev, pf_idx, fire_next):
                q0, q1 = 2 * sl, 2 * sl + 1
                wait_g(q0)
                wait_g(q1)
                if pf_idx:
                    fire_i(2 * g + 6, q0)
                    fire_i(2 * g + 7, q1)
                fire_s(g - doff, sl)
                sln = (sl + 2) % SSLICES
                qn0, qn1 = 2 * sln, 2 * sln + 1
                if wait_prev:
                    wait_s(g - doff, sln)
                if fire_next:
                    wait_i(2 * g + 4, qn0)
                    wait_i(2 * g + 5, qn1)
                    fire_g(qn0)
                    fire_g(qn1)

            for q in range(GSLOTS):
                fire_i(q, q)
            for q in range(4):
                wait_i(q, q)
                fire_g(q)
            group_body(0, 0, 0, False, True, True)

            def triple_a(p, carry):
                g = 3 * p + 1
                group_body(g, 1, 0, True, True, True)
                group_body(g + 1, 2, 0, True, True, True)
                group_body(g + 2, 0, 0, True, True, True)
                return carry

            lax.fori_loop(0, 21, triple_a, 0)

            pltpu.sync_copy(dst_hbm.at[wid, pl.ds(DHALF, GROUPS - DHALF)],
                            dst_blk.at[pl.ds(0, GROUPS - DHALF)])

            def triple_b(p, carry):
                g = 3 * p + 64
                group_body(g, 1, DHALF, True, True, True)
                group_body(g + 1, 2, DHALF, True, True, True)
                group_body(g + 2, 0, DHALF, True, True, True)
                return carry

            lax.fori_loop(0, 15, triple_b, 0)

            group_body(109, 1, DHALF, True, False, True)
            group_body(110, 2, DHALF, True, False, False)
            group_body(111, 0, DHALF, True, False, False)
            wait_s(111 - DHALF, 0)

            plsc.subcore_barrier()

            for k in range(ROWS_PER_TILE // FLUSH_CHUNK):
                off = r0 + k * FLUSH_CHUNK
                pltpu.sync_copy(acc.at[pl.ds(off, FLUSH_CHUNK)],
                                ring.at[pl.ds(0, FLUSH_CHUNK)])
                pltpu.sync_copy(ring.at[pl.ds(0, FLUSH_CHUNK)],
                                out_hbm.at[c, t, pl.ds(off, FLUSH_CHUNK)])
            plsc.subcore_barrier()

    return pl.kernel(
        body,
        out_type=jax.ShapeDtypeStruct((NUM_CORES, num_towers, NP, H),
                                      jnp.float32),
        mesh=mesh,
        scratch_types=(
            [pltpu.VMEM((GC,), jnp.int32) for _ in range(GSLOTS)]
            + [pltpu.VMEM((DHALF, SG), jnp.int32),
               pltpu.VMEM((GSLOTS * GC, H), jnp.float32),
               pltpu.VMEM_SHARED((NP, H), jnp.float32)]
            + [pltpu.SemaphoreType.DMA] * (2 * GSLOTS + SSLICES)
        ),
    )


_sc_agg_1 = _make_sc_agg(1)
_sc_agg_3 = _make_sc_agg(T)

BN = 1024
GRID = NP // BN


def _mm(a, w):
    return lax.dot_general(a, w, (((1,), (0,)), ((), ())),
                           preferred_element_type=jnp.float32,
                           precision=lax.Precision.HIGHEST)


def _mlp_first_body(scale_ref, x_ref, aggp_ref, w1_ref, b1_ref, w2_ref,
                    b2_ref, out_ref):
    agg = aggp_ref[0] + aggp_ref[1]
    x = x_ref[...]
    for t in range(T):
        u = scale_ref[t] * x + agg
        v = jnp.maximum(_mm(u, w1_ref[t]) + b1_ref[t], 0.0)
        w = jnp.maximum(_mm(v, w2_ref[t]) + b2_ref[t], 0.0)
        out_ref[t] = w


def _mlp_mid_body(scale_ref, h_ref, aggp_ref, w1_ref, b1_ref, w2_ref,
                  b2_ref, out_ref):
    for t in range(T):
        u = scale_ref[t] * h_ref[t] + (aggp_ref[0, t] + aggp_ref[1, t])
        v = jnp.maximum(_mm(u, w1_ref[t]) + b1_ref[t], 0.0)
        w = jnp.maximum(_mm(v, w2_ref[t]) + b2_ref[t], 0.0)
        out_ref[t] = w


_W_SPEC = pl.BlockSpec((T, H, H), lambda i: (0, 0, 0))
_B_SPEC = pl.BlockSpec((T, H), lambda i: (0, 0))
_H3_SPEC = pl.BlockSpec((T, BN, H), lambda i: (0, i, 0))

_mlp_first = pl.pallas_call(
    _mlp_first_body,
    grid=(GRID,),
    in_specs=[
        pl.BlockSpec(memory_space=pltpu.SMEM),
        pl.BlockSpec((BN, H), lambda i: (i, 0)),
        pl.BlockSpec((NUM_CORES, BN, H), lambda i: (0, i, 0)),
        _W_SPEC, _B_SPEC, _W_SPEC, _B_SPEC,
    ],
    out_specs=_H3_SPEC,
    out_shape=jax.ShapeDtypeStruct((T, NP, H), jnp.float32),
)

_mlp_mid = pl.pallas_call(
    _mlp_mid_body,
    grid=(GRID,),
    in_specs=[
        pl.BlockSpec(memory_space=pltpu.SMEM),
        _H3_SPEC,
        pl.BlockSpec((NUM_CORES, T, BN, H), lambda i: (0, 0, i, 0)),
        _W_SPEC, _B_SPEC, _W_SPEC, _B_SPEC,
    ],
    out_specs=_H3_SPEC,
    out_shape=jax.ShapeDtypeStruct((T, NP, H), jnp.float32),
)


def kernel(x, edge_index, W1, b1, W2, b2, eps):
    src = jnp.concatenate(
        [edge_index[0], jnp.zeros((EP - E,), jnp.int32)]
    ).reshape(NUM_WORKERS, CHUNKS, GC)
    dst = jnp.concatenate(
        [edge_index[1], jnp.full((EP - E,), N, jnp.int32)]
    ).reshape(NUM_WORKERS, GROUPS, SG)
    scale = 1.0 + eps
    zeros = jnp.zeros((FLUSH_CHUNK, H), jnp.float32)
    xp = jnp.pad(x, ((0, NP - N), (0, 0)))

    aggp0 = _sc_agg_1(xp[None], src, dst, zeros)
    h = _mlp_first(scale[:, 0], xp, aggp0[:, 0],
                   W1[:, 0], b1[:, 0], W2[:, 0], b2[:, 0])
    for l in range(1, L):
        aggp = _sc_agg_3(h, src, dst, zeros)
        h = _mlp_mid(scale[:, l], h, aggp,
                     W1[:, l], b1[:, l], W2[:, l], b2[:, l])
    return jnp.transpose(h[:, :N], (1, 0, 2))
